# baseline (speedup 1.0000x reference)
import jax
import jax.numpy as jnp
from jax.experimental import pallas as pl
from jax.experimental.pallas import tpu as pltpu
from jax.experimental.pallas import tpu_sc as plsc


_NC, _NS = 2, 16
_NW = _NC * _NS


def _sc_gather(table, ids_flat, base0, nk, chunk):
    mesh = plsc.VectorSubcoreMesh(core_axis_name="c", subcore_axis_name="s")
    b_per_w = nk // _NW
    n_chunks = b_per_w // chunk

    @pl.kernel(
        out_type=jax.ShapeDtypeStruct((nk, 128), table.dtype),
        mesh=mesh,
        scratch_types=[
            pltpu.VMEM((chunk,), jnp.int32),
            pltpu.VMEM((chunk, 128), table.dtype),
            pltpu.SemaphoreType.DMA,
        ],
    )
    def sc_kernel(table_hbm, idx_hbm, out_hbm, idx_v, rows_v, sem):
        wid = jax.lax.axis_index("s") * _NC + jax.lax.axis_index("c")
        base = wid * b_per_w

        @pl.loop(0, n_chunks)
        def _(c):
            off = base + c * chunk
            pltpu.sync_copy(idx_hbm.at[pl.ds(base0 + off, chunk)], idx_v)
            pltpu.async_copy(table_hbm.at[idx_v], rows_v, sem).wait()
            pltpu.sync_copy(rows_v, out_hbm.at[pl.ds(off, chunk)])

    return sc_kernel(table, ids_flat)


def _pad_table(table, embed_dim):
    return jnp.pad(table, ((0, 0), (0, 128 - embed_dim)))


def _tc_mlp_add_slice(
    ctx2, modal_k, w1, b1, w2, b2, out_prev, num_idx, nk, k0, embed_dim, bt
):
    ctx_dim = ctx2.shape[1]

    def body(ctx_ref, modal_ref, w1_ref, b1_ref, w2_ref, b2_ref, *rest):
        out_ref = rest[-1]
        ctx = ctx_ref[...].astype(jnp.bfloat16)
        h = jnp.dot(ctx, w1_ref[...], preferred_element_type=jnp.float32)
        h = jnp.maximum(h + b1_ref[...], 0.0).astype(jnp.bfloat16)
        y = jnp.dot(h, w2_ref[...], preferred_element_type=jnp.float32)
        modal = modal_ref[:, :embed_dim].astype(jnp.float32)
        out_ref[...] = modal + y + b2_ref[...]

    in_specs = [
        pl.BlockSpec((bt, ctx_dim), lambda i: (k0 + i, 0)),
        pl.BlockSpec((bt, 128), lambda i: (i, 0)),
        pl.BlockSpec((ctx_dim, embed_dim), lambda i: (0, 0)),
        pl.BlockSpec((1, embed_dim), lambda i: (0, 0)),
        pl.BlockSpec((embed_dim, embed_dim), lambda i: (0, 0)),
        pl.BlockSpec((1, embed_dim), lambda i: (0, 0)),
    ]
    args = [ctx2, modal_k, w1, b1, w2, b2]
    alias = {}
    if out_prev is not None:
        in_specs.append(pl.BlockSpec(memory_space=pl.ANY))
        args.append(out_prev)
        alias = {6: 0}

    return pl.pallas_call(
        body,
        grid=(nk // bt,),
        in_specs=in_specs,
        out_specs=pl.BlockSpec((bt, embed_dim), lambda i: (k0 + i, 0)),
        out_shape=jax.ShapeDtypeStruct((num_idx, embed_dim), jnp.float32),
        input_output_aliases=alias,
        compiler_params=pltpu.CompilerParams(
            dimension_semantics=("parallel",)
        ),
    )(*args)


def kernel(modality_ids, context, table, W1, b1, W2, b2):
    B, L = modality_ids.shape
    num_idx = B * L
    embed_dim = table.shape[1]
    ctx_dim = context.shape[-1]
    bt = 6400

    row_slices = [256] * 16

    ctx2 = context.reshape(num_idx, ctx_dim)
    table_p = _pad_table(table, embed_dim)
    w1 = W1.astype(jnp.bfloat16)
    w2 = W2.astype(jnp.bfloat16)
    b1r = b1.reshape(1, embed_dim)
    b2r = b2.reshape(1, embed_dim)

    ids_flat = modality_ids.reshape(num_idx).astype(jnp.int32)
    modals, bounds = [], []
    r0 = 0
    for rows in row_slices:
        nk = rows * L
        modals.append(_sc_gather(table_p, ids_flat, r0 * L, nk, chunk=400))
        bounds.append((r0 * L, nk))
        r0 += rows

    out = None
    for (tok0, nk), modal_k in zip(bounds, modals):
        out = _tc_mlp_add_slice(
            ctx2, modal_k, w1, b1r, w2, b2r, out,
            num_idx, nk, tok0 // bt, embed_dim, bt,
        )
    return out.reshape(B, L, embed_dim)


# speedup vs baseline: 1.0302x; 1.0302x over previous
import jax
import jax.numpy as jnp
from jax.experimental import pallas as pl
from jax.experimental.pallas import tpu as pltpu
from jax.experimental.pallas import tpu_sc as plsc


_NC, _NS = 2, 16
_NW = _NC * _NS


def _sc_gather(table, ids_flat, base0, nk, chunk):
    mesh = plsc.VectorSubcoreMesh(core_axis_name="c", subcore_axis_name="s")
    b_per_w = nk // _NW
    n_chunks = b_per_w // chunk

    @pl.kernel(
        out_type=jax.ShapeDtypeStruct((nk, 128), table.dtype),
        mesh=mesh,
        scratch_types=[
            pltpu.VMEM((chunk,), jnp.int32),
            pltpu.VMEM((chunk, 128), table.dtype),
            pltpu.SemaphoreType.DMA,
        ],
    )
    def sc_kernel(table_hbm, idx_hbm, out_hbm, idx_v, rows_v, sem):
        wid = jax.lax.axis_index("s") * _NC + jax.lax.axis_index("c")
        base = wid * b_per_w

        @pl.loop(0, n_chunks)
        def _(c):
            off = base + c * chunk
            pltpu.sync_copy(idx_hbm.at[pl.ds(base0 + off, chunk)], idx_v)
            pltpu.async_copy(table_hbm.at[idx_v], rows_v, sem).wait()
            pltpu.sync_copy(rows_v, out_hbm.at[pl.ds(off, chunk)])

    return sc_kernel(table, ids_flat)


def _pad_table(table, embed_dim):
    return jnp.pad(table, ((0, 0), (0, 128 - embed_dim)))


def _tc_mlp_add_slice(
    ctx2, modal_k, w1, b1, w2, b2, out_prev, num_idx, nk, k0, embed_dim, bt
):
    ctx_dim = ctx2.shape[1]

    def body(ctx_ref, modal_ref, w1_ref, b1_ref, w2_ref, b2_ref, *rest):
        out_ref = rest[-1]
        ctx = ctx_ref[...].astype(jnp.bfloat16)
        h = jnp.dot(ctx, w1_ref[...], preferred_element_type=jnp.float32)
        h = jnp.maximum(h + b1_ref[...], 0.0).astype(jnp.bfloat16)
        y = jnp.dot(h, w2_ref[...], preferred_element_type=jnp.float32)
        modal = modal_ref[:, :embed_dim].astype(jnp.float32)
        out_ref[...] = modal + y + b2_ref[...]

    in_specs = [
        pl.BlockSpec((bt, ctx_dim), lambda i: (k0 + i, 0)),
        pl.BlockSpec((bt, 128), lambda i: (i, 0)),
        pl.BlockSpec((ctx_dim, embed_dim), lambda i: (0, 0)),
        pl.BlockSpec((1, embed_dim), lambda i: (0, 0)),
        pl.BlockSpec((embed_dim, embed_dim), lambda i: (0, 0)),
        pl.BlockSpec((1, embed_dim), lambda i: (0, 0)),
    ]
    args = [ctx2, modal_k, w1, b1, w2, b2]
    alias = {}
    if out_prev is not None:
        in_specs.append(pl.BlockSpec(memory_space=pl.ANY))
        args.append(out_prev)
        alias = {6: 0}

    return pl.pallas_call(
        body,
        grid=(nk // bt,),
        in_specs=in_specs,
        out_specs=pl.BlockSpec((bt, embed_dim), lambda i: (k0 + i, 0)),
        out_shape=jax.ShapeDtypeStruct((num_idx, embed_dim), jnp.float32),
        input_output_aliases=alias,
        compiler_params=pltpu.CompilerParams(
            dimension_semantics=("parallel",)
        ),
    )(*args)


def kernel(modality_ids, context, table, W1, b1, W2, b2):
    B, L = modality_ids.shape
    num_idx = B * L
    embed_dim = table.shape[1]
    ctx_dim = context.shape[-1]
    bt = 12800

    row_slices = [512] * 8

    ctx2 = context.reshape(num_idx, ctx_dim)
    table_p = _pad_table(table, embed_dim)
    w1 = W1.astype(jnp.bfloat16)
    w2 = W2.astype(jnp.bfloat16)
    b1r = b1.reshape(1, embed_dim)
    b2r = b2.reshape(1, embed_dim)

    ids_flat = modality_ids.reshape(num_idx).astype(jnp.int32)
    modals, bounds = [], []
    r0 = 0
    for rows in row_slices:
        nk = rows * L
        modals.append(_sc_gather(table_p, ids_flat, r0 * L, nk, chunk=400))
        bounds.append((r0 * L, nk))
        r0 += rows

    out = None
    for (tok0, nk), modal_k in zip(bounds, modals):
        out = _tc_mlp_add_slice(
            ctx2, modal_k, w1, b1r, w2, b2r, out,
            num_idx, nk, tok0 // bt, embed_dim, bt,
        )
    return out.reshape(B, L, embed_dim)
